# baseline (speedup 1.0000x reference)
import functools

import jax
import jax.numpy as jnp
from jax import lax
from jax.experimental import pallas as pl
from jax.experimental.pallas import tpu as pltpu
from jax.experimental.pallas import tpu_sc as plsc

N_NODES = 10000
N_EDGES = 160000
D = 256
DH = 128
NS = 16
K = 125
CHUNKS = (N_EDGES // NS) // K
PASSES = 2
CP = CHUNKS // PASSES
NPAD = 10240
ROWS_PER_SUB = NPAD // NS


def _sc_aggregate(xa, xb, src3d, dst3d, zeros):
    mesh = plsc.VectorSubcoreMesh(core_axis_name="c", subcore_axis_name="s")

    @functools.partial(
        pl.kernel,
        out_type=(
            jax.ShapeDtypeStruct((NPAD, DH), jnp.float32),
            jax.ShapeDtypeStruct((NPAD, DH), jnp.float32),
        ),
        mesh=mesh,
        scratch_types=[
            pltpu.VMEM((CP, K), jnp.int32),
            pltpu.VMEM((CP, K), jnp.int32),
            pltpu.VMEM((K, DH), jnp.float32),
            pltpu.VMEM((K, DH), jnp.float32),
            pltpu.VMEM_SHARED((NPAD, DH), jnp.float32),
            pltpu.SemaphoreType.DMA,
            pltpu.SemaphoreType.DMA,
        ],
    )
    def agg(xa_hbm, xb_hbm, src_hbm, dst_hbm, z_hbm, oa_hbm, ob_hbm,
            src_v, dst_v, buf0, buf1, acc, sem0, sem1):
        cid = lax.axis_index("c")
        sid = lax.axis_index("s")

        slab = pl.ds(sid * ROWS_PER_SUB, ROWS_PER_SUB)

        pltpu.sync_copy(z_hbm, acc.at[slab])

        plsc.subcore_barrier()

        def run(x_hbm):
            def start(j, buf, sem):
                pltpu.async_copy(x_hbm.at[src_v.at[j]], buf, sem)

            def finish(buf, sem):
                pltpu.make_async_copy(x_hbm.at[src_v.at[0]], buf, sem).wait()

            def scat(j, buf):
                pltpu.sync_copy(buf, acc.at[dst_v.at[j]], add=True)

            for p in range(PASSES):
                v = sid * PASSES + p
                pltpu.sync_copy(src_hbm.at[v], src_v)
                pltpu.sync_copy(dst_hbm.at[v], dst_v)

                start(0, buf0, sem0)

                @pl.loop(0, CP - 2, step=2)
                def _(j):
                    start(j + 1, buf1, sem1)
                    finish(buf0, sem0)
                    scat(j, buf0)
                    start(j + 2, buf0, sem0)
                    finish(buf1, sem1)
                    scat(j + 1, buf1)

                start(CP - 1, buf1, sem1)
                finish(buf0, sem0)
                scat(CP - 2, buf0)
                finish(buf1, sem1)
                scat(CP - 1, buf1)

        @pl.when(cid == 0)
        def _():
            run(xa_hbm)

        @pl.when(cid == 1)
        def _():
            run(xb_hbm)

        plsc.subcore_barrier()

        @pl.when(cid == 0)
        def _():
            pltpu.sync_copy(acc.at[slab], oa_hbm.at[slab])

        @pl.when(cid == 1)
        def _():
            pltpu.sync_copy(acc.at[slab], ob_hbm.at[slab])

    return agg(xa, xb, src3d, dst3d, zeros)


def _tc_split(x):
    BM = 2000

    def sp(x_ref, a_ref, b_ref):
        a_ref[...] = x_ref[:, :DH]
        b_ref[...] = x_ref[:, DH:]

    return pl.pallas_call(
        sp,
        grid=(N_NODES // BM,),
        in_specs=[pl.BlockSpec((BM, D), lambda i: (i, 0))],
        out_specs=[
            pl.BlockSpec((BM, DH), lambda i: (i, 0)),
            pl.BlockSpec((BM, DH), lambda i: (i, 0)),
        ],
        out_shape=[
            jax.ShapeDtypeStruct((N_NODES, DH), jnp.float32),
            jax.ShapeDtypeStruct((N_NODES, DH), jnp.float32),
        ],
    )(x)


def _tc_matmul(oa, ob, wta, wtb):
    BM = 1000

    def mm(a_ref, b_ref, wa_ref, wb_ref, o_ref):
        a16 = a_ref[...].astype(jnp.bfloat16)
        b16 = b_ref[...].astype(jnp.bfloat16)
        o_ref[...] = (
            jnp.dot(a16, wa_ref[...], preferred_element_type=jnp.float32)
            + jnp.dot(b16, wb_ref[...], preferred_element_type=jnp.float32))

    return pl.pallas_call(
        mm,
        grid=(N_NODES // BM,),
        in_specs=[
            pl.BlockSpec((BM, DH), lambda i: (i, 0)),
            pl.BlockSpec((BM, DH), lambda i: (i, 0)),
            pl.BlockSpec((DH, D), lambda i: (0, 0)),
            pl.BlockSpec((DH, D), lambda i: (0, 0)),
        ],
        out_specs=pl.BlockSpec((BM, D), lambda i: (i, 0)),
        out_shape=jax.ShapeDtypeStruct((N_NODES, D), jnp.float32),
    )(oa, ob, wta, wtb)


def kernel(x, edge_index, W):
    xa, xb = _tc_split(x)
    src3d = edge_index[0].reshape(NS * PASSES, CP, K)
    dst3d = edge_index[1].reshape(NS * PASSES, CP, K)
    zeros = jnp.zeros((ROWS_PER_SUB, DH), jnp.float32)
    oa, ob = _sc_aggregate(xa, xb, src3d, dst3d, zeros)
    wt = W.T.astype(jnp.bfloat16)
    return _tc_matmul(oa, ob, wt[:DH], wt[DH:])


# speedup vs baseline: 1.0288x; 1.0288x over previous
import functools

import jax
import jax.numpy as jnp
from jax import lax
from jax.experimental import pallas as pl
from jax.experimental.pallas import tpu as pltpu
from jax.experimental.pallas import tpu_sc as plsc

N_NODES = 10000
N_EDGES = 160000
D = 256
DH = 128
NS = 16
K = 125
CHUNKS = (N_EDGES // NS) // K
PASSES = 2
CP = CHUNKS // PASSES
NPAD = 10240
ROWS_PER_SUB = NPAD // NS


def _sc_aggregate(xa, xb, src3d, dst3d, zeros):
    mesh = plsc.VectorSubcoreMesh(core_axis_name="c", subcore_axis_name="s")

    @functools.partial(
        pl.kernel,
        out_type=(
            jax.ShapeDtypeStruct((NPAD, DH), jnp.float32),
            jax.ShapeDtypeStruct((NPAD, DH), jnp.float32),
        ),
        mesh=mesh,
        scratch_types=[
            pltpu.VMEM((CP, K), jnp.int32),
            pltpu.VMEM((CP, K), jnp.int32),
            pltpu.VMEM((K, DH), jnp.float32),
            pltpu.VMEM((K, DH), jnp.float32),
            pltpu.VMEM_SHARED((NPAD, DH), jnp.float32),
            pltpu.SemaphoreType.DMA,
            pltpu.SemaphoreType.DMA,
        ],
    )
    def agg(xa_hbm, xb_hbm, src_hbm, dst_hbm, z_hbm, oa_hbm, ob_hbm,
            src_v, dst_v, buf0, buf1, acc, sem0, sem1):
        cid = lax.axis_index("c")
        sid = lax.axis_index("s")

        slab = pl.ds(sid * ROWS_PER_SUB, ROWS_PER_SUB)

        pltpu.sync_copy(z_hbm, acc.at[slab])

        plsc.subcore_barrier()

        def run(x_hbm):
            def start(j, buf, sem):
                pltpu.async_copy(x_hbm.at[src_v.at[j]], buf, sem)

            def finish(buf, sem):
                pltpu.make_async_copy(x_hbm.at[src_v.at[0]], buf, sem).wait()

            def scat(j, buf):
                pltpu.sync_copy(buf, acc.at[dst_v.at[j]], add=True)

            for p in range(PASSES):
                v = sid * PASSES + p
                pltpu.sync_copy(src_hbm.at[v], src_v)
                pltpu.sync_copy(dst_hbm.at[v], dst_v)

                start(0, buf0, sem0)

                @pl.loop(0, CP - 2, step=2)
                def _(j):
                    start(j + 1, buf1, sem1)
                    finish(buf0, sem0)
                    scat(j, buf0)
                    start(j + 2, buf0, sem0)
                    finish(buf1, sem1)
                    scat(j + 1, buf1)

                start(CP - 1, buf1, sem1)
                finish(buf0, sem0)
                scat(CP - 2, buf0)
                finish(buf1, sem1)
                scat(CP - 1, buf1)

        @pl.when(cid == 0)
        def _():
            run(xa_hbm)

        @pl.when(cid == 1)
        def _():
            run(xb_hbm)

        plsc.subcore_barrier()

        @pl.when(cid == 0)
        def _():
            pltpu.sync_copy(acc.at[slab], oa_hbm.at[slab])

        @pl.when(cid == 1)
        def _():
            pltpu.sync_copy(acc.at[slab], ob_hbm.at[slab])

    return agg(xa, xb, src3d, dst3d, zeros)


def _tc_matmul(oa, ob, wta, wtb):
    BM = 2000

    def mm(a_ref, b_ref, wa_ref, wb_ref, o_ref):
        a16 = a_ref[...].astype(jnp.bfloat16)
        b16 = b_ref[...].astype(jnp.bfloat16)
        o_ref[...] = (
            jnp.dot(a16, wa_ref[...], preferred_element_type=jnp.float32)
            + jnp.dot(b16, wb_ref[...], preferred_element_type=jnp.float32))

    return pl.pallas_call(
        mm,
        grid=(N_NODES // BM,),
        in_specs=[
            pl.BlockSpec((BM, DH), lambda i: (i, 0)),
            pl.BlockSpec((BM, DH), lambda i: (i, 0)),
            pl.BlockSpec((DH, D), lambda i: (0, 0)),
            pl.BlockSpec((DH, D), lambda i: (0, 0)),
        ],
        out_specs=pl.BlockSpec((BM, D), lambda i: (i, 0)),
        out_shape=jax.ShapeDtypeStruct((N_NODES, D), jnp.float32),
    )(oa, ob, wta, wtb)


def kernel(x, edge_index, W):
    xa = x[:, :DH]
    xb = x[:, DH:]
    src3d = edge_index[0].reshape(NS * PASSES, CP, K)
    dst3d = edge_index[1].reshape(NS * PASSES, CP, K)
    zeros = jnp.zeros((ROWS_PER_SUB, DH), jnp.float32)
    oa, ob = _sc_aggregate(xa, xb, src3d, dst3d, zeros)
    wt = W.T.astype(jnp.bfloat16)
    return _tc_matmul(oa, ob, wt[:DH], wt[DH:])


# speedup vs baseline: 1.0387x; 1.0096x over previous
import functools

import jax
import jax.numpy as jnp
from jax import lax
from jax.experimental import pallas as pl
from jax.experimental.pallas import tpu as pltpu
from jax.experimental.pallas import tpu_sc as plsc

N_NODES = 10000
N_EDGES = 160000
D = 256
DH = 128
NS = 16
K = 125
CHUNKS = (N_EDGES // NS) // K
PASSES = 2
CP = CHUNKS // PASSES
NPAD = 10240
ROWS_PER_SUB = NPAD // NS


def _sc_aggregate(xa, xb, src3d, dst3d, zeros):
    mesh = plsc.VectorSubcoreMesh(core_axis_name="c", subcore_axis_name="s")

    @functools.partial(
        pl.kernel,
        out_type=(
            jax.ShapeDtypeStruct((NPAD, DH), jnp.float32),
            jax.ShapeDtypeStruct((NPAD, DH), jnp.float32),
        ),
        mesh=mesh,
        scratch_types=[
            pltpu.VMEM((CP, K), jnp.int32),
            pltpu.VMEM((CP, K), jnp.int32),
            pltpu.VMEM((K, DH), jnp.float32),
            pltpu.VMEM((K, DH), jnp.float32),
            pltpu.VMEM_SHARED((NPAD, DH), jnp.float32),
            pltpu.SemaphoreType.DMA,
            pltpu.SemaphoreType.DMA,
        ],
    )
    def agg(xa_hbm, xb_hbm, src_hbm, dst_hbm, z_hbm, oa_hbm, ob_hbm,
            src_v, dst_v, buf0, buf1, acc, sem0, sem1):
        cid = lax.axis_index("c")
        sid = lax.axis_index("s")

        slab = pl.ds(sid * ROWS_PER_SUB, ROWS_PER_SUB)

        pltpu.sync_copy(z_hbm, acc.at[slab])

        def run(x_hbm):
            def start(j, buf, sem):
                pltpu.async_copy(x_hbm.at[src_v.at[j]], buf, sem)

            def finish(buf, sem):
                pltpu.make_async_copy(x_hbm.at[src_v.at[0]], buf, sem).wait()

            def scat(j, buf):
                pltpu.sync_copy(buf, acc.at[dst_v.at[j]], add=True)

            for p in range(PASSES):
                v = sid * PASSES + p
                pltpu.sync_copy(src_hbm.at[v], src_v)
                pltpu.sync_copy(dst_hbm.at[v], dst_v)

                start(0, buf0, sem0)
                if p == 0:
                    plsc.subcore_barrier()

                @pl.loop(0, CP - 2, step=2)
                def _(j):
                    start(j + 1, buf1, sem1)
                    finish(buf0, sem0)
                    scat(j, buf0)
                    start(j + 2, buf0, sem0)
                    finish(buf1, sem1)
                    scat(j + 1, buf1)

                start(CP - 1, buf1, sem1)
                finish(buf0, sem0)
                scat(CP - 2, buf0)
                finish(buf1, sem1)
                scat(CP - 1, buf1)

        @pl.when(cid == 0)
        def _():
            run(xa_hbm)

        @pl.when(cid == 1)
        def _():
            run(xb_hbm)

        plsc.subcore_barrier()

        @pl.when(cid == 0)
        def _():
            pltpu.sync_copy(acc.at[slab], oa_hbm.at[slab])

        @pl.when(cid == 1)
        def _():
            pltpu.sync_copy(acc.at[slab], ob_hbm.at[slab])

    return agg(xa, xb, src3d, dst3d, zeros)


def _tc_matmul(oa, ob, wta, wtb):
    BM = 2000

    def mm(a_ref, b_ref, wa_ref, wb_ref, o_ref):
        a16 = a_ref[...].astype(jnp.bfloat16)
        b16 = b_ref[...].astype(jnp.bfloat16)
        o_ref[...] = (
            jnp.dot(a16, wa_ref[...], preferred_element_type=jnp.float32)
            + jnp.dot(b16, wb_ref[...], preferred_element_type=jnp.float32))

    return pl.pallas_call(
        mm,
        grid=(N_NODES // BM,),
        in_specs=[
            pl.BlockSpec((BM, DH), lambda i: (i, 0)),
            pl.BlockSpec((BM, DH), lambda i: (i, 0)),
            pl.BlockSpec((DH, D), lambda i: (0, 0)),
            pl.BlockSpec((DH, D), lambda i: (0, 0)),
        ],
        out_specs=pl.BlockSpec((BM, D), lambda i: (i, 0)),
        out_shape=jax.ShapeDtypeStruct((N_NODES, D), jnp.float32),
    )(oa, ob, wta, wtb)


def kernel(x, edge_index, W):
    xa = x[:, :DH]
    xb = x[:, DH:]
    src3d = edge_index[0].reshape(NS * PASSES, CP, K)
    dst3d = edge_index[1].reshape(NS * PASSES, CP, K)
    zeros = jnp.zeros((ROWS_PER_SUB, DH), jnp.float32)
    oa, ob = _sc_aggregate(xa, xb, src3d, dst3d, zeros)
    wt = W.T.astype(jnp.bfloat16)
    return _tc_matmul(oa, ob, wt[:DH], wt[DH:])
